# Initial kernel scaffold; baseline (speedup 1.0000x reference)
import jax
import jax.numpy as jnp
from jax.experimental import pallas as pl


def kernel(output1, output2, label):
    raise NotImplementedError("write your pallas kernel here")


# speedup vs baseline: 2.2853x; 2.2853x over previous
import functools

import jax
import jax.numpy as jnp
from jax import lax
from jax.experimental import pallas as pl
from jax.experimental.pallas import tpu as pltpu

N = 16384
D = 128
K = N // 2
MARGIN = 2.0
EPS = 1e-6

_RB = 16


def _dense_body(o1_ref, o2_ref, lab_ref, out_ref):
    d = o1_ref[...] - o2_ref[...] + EPS
    s = jnp.sum(d * d, axis=2)
    dist = jnp.sqrt(s)
    labf = lab_ref[...].astype(jnp.float32)
    hinge = jnp.maximum(MARGIN - dist, 0.0)
    out_ref[...] = labf * s + (1.0 - labf) * hinge * hinge


def _select_body(loss_ref, out_ref):
    loss = loss_ref[...]
    bits = lax.bitcast_convert_type(loss, jnp.int32)

    def round_fn(j, lo):
        mid = lo + jnp.left_shift(jnp.int32(1), 30 - j)
        cnt = jnp.sum((bits >= mid).astype(jnp.int32))
        return jnp.where(cnt >= K, mid, lo)

    lo = lax.fori_loop(0, 31, round_fn, jnp.int32(0))
    t = lax.bitcast_convert_type(lo, jnp.float32)
    gt = bits > lo
    cnt_gt = jnp.sum(gt.astype(jnp.float32))
    sum_gt = jnp.sum(jnp.where(gt, loss, 0.0))
    res = (sum_gt + (jnp.float32(K) - cnt_gt) * t) * (1.0 / K)
    out_ref[...] = jnp.full((1, 1), res, jnp.float32)


def kernel(output1, output2, label):
    o1 = output1.reshape(N // D, D, D)
    o2 = output2.reshape(N // D, D, D)
    lab = label.astype(jnp.int32).reshape(N // D, D)

    grid = (N // D) // _RB
    loss = pl.pallas_call(
        _dense_body,
        grid=(grid,),
        in_specs=[
            pl.BlockSpec((_RB, D, D), lambda i: (i, 0, 0)),
            pl.BlockSpec((_RB, D, D), lambda i: (i, 0, 0)),
            pl.BlockSpec((_RB, D), lambda i: (i, 0)),
        ],
        out_specs=pl.BlockSpec((_RB, D), lambda i: (i, 0)),
        out_shape=jax.ShapeDtypeStruct((N // D, D), jnp.float32),
    )(o1, o2, lab)

    out = pl.pallas_call(
        _select_body,
        out_shape=jax.ShapeDtypeStruct((1, 1), jnp.float32),
    )(loss)
    return out[0, 0]
